# baseline (speedup 1.0000x reference)
import functools
import jax
import jax.numpy as jnp
from jax.experimental import pallas as pl
from jax.experimental.pallas import tpu as pltpu

_B, _T, _C = 1, 512, 256
_W = 512
_ER, _ES, _K = 16, 2, 2
_EPS = 1.1920929e-07
_EPG = 4


def _rms(x, g):
    return x * jax.lax.rsqrt(jnp.mean(x * x, axis=-1, keepdims=True) + _EPS) * g


def _gelu(x):
    return 0.5 * x * (1.0 + jax.lax.erf(x * 0.7071067811865476))


def _moe_body(u_ref, cent_ref, sg_ref, rg_ref,
              sW1_ref, sb1_ref, sW2_ref, sb2_ref,
              rW1_ref, rb1_ref, rW2_ref, rb2_ref,
              out_ref, g_scr, xnb_scr, h_scr, hs_scr):
    e = pl.program_id(0)
    ph = pl.program_id(1)
    ids = jax.lax.broadcasted_iota(jnp.int32, (_T, _ER), 1)
    bf = jnp.bfloat16

    @pl.when(jnp.logical_and(e == 0, ph == 0))
    def _init():
        u = u_ref[...]
        s = jax.nn.sigmoid(
            jnp.dot(u, cent_ref[...], preferred_element_type=jnp.float32))
        denom = jnp.sum(s, axis=1, keepdims=True)
        m1 = jnp.max(s, axis=1, keepdims=True)
        i1 = jnp.min(jnp.where(s == m1, ids, _ER), axis=1, keepdims=True)
        s2 = jnp.where(ids == i1, -jnp.inf, s)
        m2 = jnp.max(s2, axis=1, keepdims=True)
        i2 = jnp.min(jnp.where(s2 == m2, ids, _ER), axis=1, keepdims=True)
        g_scr[...] = (jnp.where(ids == i1, m1 / denom, 0.0)
                      + jnp.where(ids == i2, m2 / denom, 0.0))
        xnb_scr[...] = _rms(u, rg_ref[0, :]).astype(bf)
        out_ref[...] = u

    xnb = xnb_scr[...]

    @pl.when(ph == 0)
    def _up():
        @pl.when(e < _ES)
        def _shared_up():
            xns = _rms(u_ref[...], sg_ref[0, :])
            hs_scr[...] = _gelu(
                (jnp.dot(xns.astype(bf), sW1_ref[0].astype(bf),
                         preferred_element_type=jnp.float32)
                 + sb1_ref[0]).astype(bf))

        for sub in range(_EPG):
            h_scr[sub] = _gelu(
                (jnp.dot(xnb, rW1_ref[sub].astype(bf),
                         preferred_element_type=jnp.float32)
                 + rb1_ref[sub]).astype(bf))

    @pl.when(ph == 1)
    def _down():
        acc = jnp.zeros((_T, _C), jnp.float32)
        for sub in range(_EPG):
            ee = e * _EPG + sub
            y = jnp.dot(h_scr[sub], rW2_ref[sub].astype(bf),
                        preferred_element_type=jnp.float32) + rb2_ref[sub]
            gcol = jnp.sum(jnp.where(ids == ee, g_scr[...], 0.0), axis=1,
                           keepdims=True)
            acc = acc + gcol * y

        @pl.when(e < _ES)
        def _shared_down():
            acc2 = (jnp.dot(hs_scr[...], sW2_ref[0].astype(bf),
                            preferred_element_type=jnp.float32)
                    + sb2_ref[0])
            out_ref[...] += acc + acc2

        @pl.when(e >= _ES)
        def _routed_only():
            out_ref[...] += acc


def kernel(u, shared_W1, shared_b1, shared_W2, shared_b2, shared_g,
           routed_W1, routed_b1, routed_W2, routed_b2, routed_g, centroids):
    u2 = u.reshape(_T, _C)
    out = pl.pallas_call(
        _moe_body,
        grid=(_ER // _EPG, 2),
        in_specs=[
            pl.BlockSpec((_T, _C), lambda e, p: (0, 0)),
            pl.BlockSpec((_C, _ER), lambda e, p: (0, 0)),
            pl.BlockSpec((1, _C), lambda e, p: (0, 0)),
            pl.BlockSpec((1, _C), lambda e, p: (0, 0)),
            pl.BlockSpec((1, _C, _W),
                         lambda e, p: (jnp.minimum(e, _ES - 1), 0, 0)),
            pl.BlockSpec((1, 1, _W),
                         lambda e, p: (jnp.minimum(e, _ES - 1), 0, 0)),
            pl.BlockSpec((1, _W, _C),
                         lambda e, p: (jnp.minimum(e, _ES - 1), 0, 0)),
            pl.BlockSpec((1, 1, _C),
                         lambda e, p: (jnp.minimum(e, _ES - 1), 0, 0)),
            pl.BlockSpec((_EPG, _C, _W), lambda e, p: (e, 0, 0)),
            pl.BlockSpec((_EPG, 1, _W), lambda e, p: (e, 0, 0)),
            pl.BlockSpec((_EPG, _W, _C), lambda e, p: (e, 0, 0)),
            pl.BlockSpec((_EPG, 1, _C), lambda e, p: (e, 0, 0)),
        ],
        out_specs=pl.BlockSpec((_T, _C), lambda e, p: (0, 0)),
        out_shape=jax.ShapeDtypeStruct((_T, _C), jnp.float32),
        scratch_shapes=[
            pltpu.VMEM((_T, _ER), jnp.float32),
            pltpu.VMEM((_T, _C), jnp.bfloat16),
            pltpu.VMEM((_EPG, _T, _W), jnp.bfloat16),
            pltpu.VMEM((_T, _W), jnp.bfloat16),
        ],
        compiler_params=pltpu.CompilerParams(
            dimension_semantics=("arbitrary", "arbitrary"),
        ),
    )(
        u2, centroids,
        shared_g.reshape(1, _C), routed_g.reshape(1, _C),
        shared_W1, shared_b1.reshape(_ES, 1, _W),
        shared_W2, shared_b2.reshape(_ES, 1, _C),
        routed_W1, routed_b1.reshape(_ER, 1, _W),
        routed_W2, routed_b2.reshape(_ER, 1, _C),
    )
    return out.reshape(_B, _T, _C)


# speedup vs baseline: 1.1487x; 1.1487x over previous
import functools
import jax
import jax.numpy as jnp
from jax.experimental import pallas as pl
from jax.experimental.pallas import tpu as pltpu

_B, _T, _C = 1, 512, 256
_W = 512
_ER, _ES, _K = 16, 2, 2
_EPS = 1.1920929e-07


def _rms(x, g):
    return x * jax.lax.rsqrt(jnp.mean(x * x, axis=-1, keepdims=True) + _EPS) * g


def _gelu(x):
    return 0.5 * x * (1.0 + jax.lax.erf(x * 0.7071067811865476))


def _dense_body(u_ref, cent_ref, sg_ref, rg_ref,
                sW1_ref, sb1_ref, sW2_ref, sb2_ref,
                rW1_ref, rb1_ref, rW2_ref, rb2_ref,
                out_ref, g_scr):
    e = pl.program_id(0)
    u = u_ref[...]
    ids = jax.lax.broadcasted_iota(jnp.int32, (_T, _ER), 1)

    @pl.when(e == 0)
    def _init():
        s = jax.nn.sigmoid(
            jnp.dot(u, cent_ref[...], preferred_element_type=jnp.float32))
        denom = jnp.sum(s, axis=1, keepdims=True)
        m1 = jnp.max(s, axis=1, keepdims=True)
        i1 = jnp.min(jnp.where(s == m1, ids, _ER), axis=1, keepdims=True)
        s2 = jnp.where(ids == i1, -jnp.inf, s)
        m2 = jnp.max(s2, axis=1, keepdims=True)
        i2 = jnp.min(jnp.where(s2 == m2, ids, _ER), axis=1, keepdims=True)
        gfull = (jnp.where(ids == i1, m1 / denom, 0.0)
                 + jnp.where(ids == i2, m2 / denom, 0.0))
        g_scr[...] = gfull
        out_ref[...] = u

    bf = jnp.bfloat16

    @pl.when(e < _ES)
    def _shared():
        xn = _rms(u, sg_ref[0, :])
        h = _gelu((jnp.dot(xn.astype(bf), sW1_ref[0].astype(bf),
                           preferred_element_type=jnp.float32)
                   + sb1_ref[0]).astype(bf))
        out_ref[...] += (jnp.dot(h, sW2_ref[0].astype(bf),
                                 preferred_element_type=jnp.float32)
                         + sb2_ref[0])

    xn = _rms(u, rg_ref[0, :])
    xnb = xn.astype(bf)
    acc = jnp.zeros((_T, _C), jnp.float32)
    for sub in range(4):
        ee = e * 4 + sub
        h = _gelu((jnp.dot(xnb, rW1_ref[sub].astype(bf),
                           preferred_element_type=jnp.float32)
                   + rb1_ref[sub]).astype(bf))
        y = jnp.dot(h, rW2_ref[sub].astype(bf),
                    preferred_element_type=jnp.float32) + rb2_ref[sub]
        gcol = jnp.sum(jnp.where(ids == ee, g_scr[...], 0.0), axis=1,
                       keepdims=True)
        acc = acc + gcol * y
    out_ref[...] += acc


def kernel(u, shared_W1, shared_b1, shared_W2, shared_b2, shared_g,
           routed_W1, routed_b1, routed_W2, routed_b2, routed_g, centroids):
    u2 = u.reshape(_T, _C)
    out = pl.pallas_call(
        _dense_body,
        grid=(_ER // 4,),
        in_specs=[
            pl.BlockSpec((_T, _C), lambda e: (0, 0)),
            pl.BlockSpec((_C, _ER), lambda e: (0, 0)),
            pl.BlockSpec((1, _C), lambda e: (0, 0)),
            pl.BlockSpec((1, _C), lambda e: (0, 0)),
            pl.BlockSpec((1, _C, _W), lambda e: (jnp.minimum(e, _ES - 1), 0, 0)),
            pl.BlockSpec((1, 1, _W), lambda e: (jnp.minimum(e, _ES - 1), 0, 0)),
            pl.BlockSpec((1, _W, _C), lambda e: (jnp.minimum(e, _ES - 1), 0, 0)),
            pl.BlockSpec((1, 1, _C), lambda e: (jnp.minimum(e, _ES - 1), 0, 0)),
            pl.BlockSpec((4, _C, _W), lambda e: (e, 0, 0)),
            pl.BlockSpec((4, 1, _W), lambda e: (e, 0, 0)),
            pl.BlockSpec((4, _W, _C), lambda e: (e, 0, 0)),
            pl.BlockSpec((4, 1, _C), lambda e: (e, 0, 0)),
        ],
        out_specs=pl.BlockSpec((_T, _C), lambda e: (0, 0)),
        out_shape=jax.ShapeDtypeStruct((_T, _C), jnp.float32),
        scratch_shapes=[pltpu.VMEM((_T, _ER), jnp.float32)],
        compiler_params=pltpu.CompilerParams(
            dimension_semantics=("arbitrary",),
        ),
    )(
        u2, centroids,
        shared_g.reshape(1, _C), routed_g.reshape(1, _C),
        shared_W1, shared_b1.reshape(_ES, 1, _W),
        shared_W2, shared_b2.reshape(_ES, 1, _C),
        routed_W1, routed_b1.reshape(_ER, 1, _W),
        routed_W2, routed_b2.reshape(_ER, 1, _C),
    )
    return out.reshape(_B, _T, _C)


# speedup vs baseline: 1.1611x; 1.0108x over previous
import functools
import jax
import jax.numpy as jnp
from jax.experimental import pallas as pl
from jax.experimental.pallas import tpu as pltpu

_B, _T, _C = 1, 512, 256
_W = 512
_ER, _ES, _K = 16, 2, 2
_EPS = 1.1920929e-07


def _rms(x, g):
    return x * jax.lax.rsqrt(jnp.mean(x * x, axis=-1, keepdims=True) + _EPS) * g


def _gelu(x):
    return 0.5 * x * (1.0 + jax.lax.erf(x * 0.7071067811865476))


def _dense_body(u_ref, cent_ref, sg_ref, rg_ref,
                sW1_ref, sb1_ref, sW2_ref, sb2_ref,
                rW1_ref, rb1_ref, rW2_ref, rb2_ref,
                out_ref, g_scr, xnb_scr):
    e = pl.program_id(0)
    u = u_ref[...]
    ids = jax.lax.broadcasted_iota(jnp.int32, (_T, _ER), 1)
    bf = jnp.bfloat16

    @pl.when(e == 0)
    def _init():
        s = jax.nn.sigmoid(
            jnp.dot(u, cent_ref[...], preferred_element_type=jnp.float32))
        denom = jnp.sum(s, axis=1, keepdims=True)
        m1 = jnp.max(s, axis=1, keepdims=True)
        i1 = jnp.min(jnp.where(s == m1, ids, _ER), axis=1, keepdims=True)
        s2 = jnp.where(ids == i1, -jnp.inf, s)
        m2 = jnp.max(s2, axis=1, keepdims=True)
        i2 = jnp.min(jnp.where(s2 == m2, ids, _ER), axis=1, keepdims=True)
        gfull = (jnp.where(ids == i1, m1 / denom, 0.0)
                 + jnp.where(ids == i2, m2 / denom, 0.0))
        g_scr[...] = gfull
        xnb_scr[...] = _rms(u, rg_ref[0, :]).astype(bf)
        out_ref[...] = u

    @pl.when(e < _ES)
    def _shared():
        xn = _rms(u, sg_ref[0, :])
        h = _gelu((jnp.dot(xn.astype(bf), sW1_ref[0].astype(bf),
                           preferred_element_type=jnp.float32)
                   + sb1_ref[0]).astype(bf))
        out_ref[...] += (jnp.dot(h, sW2_ref[0].astype(bf),
                                 preferred_element_type=jnp.float32)
                         + sb2_ref[0])

    xnb = xnb_scr[...]
    acc = jnp.zeros((_T, _C), jnp.float32)
    for sub in range(4):
        ee = e * 4 + sub
        h = _gelu((jnp.dot(xnb, rW1_ref[sub].astype(bf),
                           preferred_element_type=jnp.float32)
                   + rb1_ref[sub]).astype(bf))
        y = jnp.dot(h, rW2_ref[sub].astype(bf),
                    preferred_element_type=jnp.float32) + rb2_ref[sub]
        gcol = jnp.sum(jnp.where(ids == ee, g_scr[...], 0.0), axis=1,
                       keepdims=True)
        acc = acc + gcol * y
    out_ref[...] += acc


def kernel(u, shared_W1, shared_b1, shared_W2, shared_b2, shared_g,
           routed_W1, routed_b1, routed_W2, routed_b2, routed_g, centroids):
    u2 = u.reshape(_T, _C)
    out = pl.pallas_call(
        _dense_body,
        grid=(_ER // 4,),
        in_specs=[
            pl.BlockSpec((_T, _C), lambda e: (0, 0)),
            pl.BlockSpec((_C, _ER), lambda e: (0, 0)),
            pl.BlockSpec((1, _C), lambda e: (0, 0)),
            pl.BlockSpec((1, _C), lambda e: (0, 0)),
            pl.BlockSpec((1, _C, _W), lambda e: (jnp.minimum(e, _ES - 1), 0, 0)),
            pl.BlockSpec((1, 1, _W), lambda e: (jnp.minimum(e, _ES - 1), 0, 0)),
            pl.BlockSpec((1, _W, _C), lambda e: (jnp.minimum(e, _ES - 1), 0, 0)),
            pl.BlockSpec((1, 1, _C), lambda e: (jnp.minimum(e, _ES - 1), 0, 0)),
            pl.BlockSpec((4, _C, _W), lambda e: (e, 0, 0)),
            pl.BlockSpec((4, 1, _W), lambda e: (e, 0, 0)),
            pl.BlockSpec((4, _W, _C), lambda e: (e, 0, 0)),
            pl.BlockSpec((4, 1, _C), lambda e: (e, 0, 0)),
        ],
        out_specs=pl.BlockSpec((_T, _C), lambda e: (0, 0)),
        out_shape=jax.ShapeDtypeStruct((_T, _C), jnp.float32),
        scratch_shapes=[pltpu.VMEM((_T, _ER), jnp.float32),
                        pltpu.VMEM((_T, _C), jnp.bfloat16)],
        compiler_params=pltpu.CompilerParams(
            dimension_semantics=("arbitrary",),
        ),
    )(
        u2, centroids,
        shared_g.reshape(1, _C), routed_g.reshape(1, _C),
        shared_W1, shared_b1.reshape(_ES, 1, _W),
        shared_W2, shared_b2.reshape(_ES, 1, _C),
        routed_W1, routed_b1.reshape(_ER, 1, _W),
        routed_W2, routed_b2.reshape(_ER, 1, _C),
    )
    return out.reshape(_B, _T, _C)
